# baseline (speedup 1.0000x reference)
import functools

import jax
import jax.numpy as jnp
from jax.experimental import pallas as pl
from jax.experimental.pallas import tpu as pltpu

N = 10000
F = 128
K = 10
NUM_STEPS = 2

MBLK = 200
NPAD = 10112
NCHUNK = NPAD // 128
NEG = -3.0e38


def _topk_body(hs_ref, ht_ref, idx_ref, s_ref):
    s = jax.lax.dot_general(hs_ref[...], ht_ref[...], (((1,), (1,)), ((), ())),
                            preferred_element_type=jnp.float32)
    col = jax.lax.broadcasted_iota(jnp.int32, (MBLK, NPAD), 1)
    s_ref[...] = jnp.where(col < N, s, NEG)

    def max_tile(t, m):
        tile = s_ref[:, pl.ds(t * 128, 128)]
        return jnp.maximum(m, jnp.max(tile, axis=1, keepdims=True))

    def loc_tile(t, best):
        tile = s_ref[:, pl.ds(t * 128, 128)]
        ti = jax.lax.broadcasted_iota(jnp.int32, (MBLK, 128), 1) + t * 128
        m, _ = best
        cand = jnp.min(jnp.where(tile == m, ti, jnp.int32(2**30)), axis=1,
                       keepdims=True)
        return (m, jnp.minimum(best[1], cand))

    def mask_tile(t, idx):
        tile = s_ref[:, pl.ds(t * 128, 128)]
        ti = jax.lax.broadcasted_iota(jnp.int32, (MBLK, 128), 1) + t * 128
        s_ref[:, pl.ds(t * 128, 128)] = jnp.where(ti == idx, NEG, tile)
        return idx

    cols = []
    for _ in range(K):
        m = jax.lax.fori_loop(0, NCHUNK, max_tile,
                              jnp.full((MBLK, 1), NEG, jnp.float32))
        _, idx = jax.lax.fori_loop(0, NCHUNK, loc_tile,
                                   (m, jnp.full((MBLK, 1), 2**30, jnp.int32)))
        idx = jax.lax.fori_loop(0, NCHUNK, mask_tile, idx)
        cols.append(idx)
    idx_ref[...] = jnp.concatenate(cols, axis=1)


def _topk_idx(h_s_bf, h_t_bf):
    ht_pad = jnp.zeros((NPAD, F), jnp.bfloat16).at[:N].set(h_t_bf)
    return pl.pallas_call(
        _topk_body,
        grid=(N // MBLK,),
        in_specs=[
            pl.BlockSpec((MBLK, F), lambda i: (i, 0)),
            pl.BlockSpec((NPAD, F), lambda i: (0, 0)),
        ],
        out_specs=pl.BlockSpec((MBLK, K), lambda i: (i, 0)),
        out_shape=jax.ShapeDtypeStruct((N, K), jnp.int32),
        scratch_shapes=[pltpu.VMEM((MBLK, NPAD), jnp.float32)],
    )(h_s_bf, ht_pad)


def _softmax2_body(a_ref, b_ref, oa_ref, ob_ref):
    a = a_ref[...]
    b = b_ref[...]
    a = a - jnp.max(a, axis=-1, keepdims=True)
    ea = jnp.exp(a)
    oa_ref[...] = ea / jnp.sum(ea, axis=-1, keepdims=True)
    b = b - jnp.max(b, axis=-1, keepdims=True)
    eb = jnp.exp(b)
    ob_ref[...] = eb / jnp.sum(eb, axis=-1, keepdims=True)


def _softmax_pair(s_hat0, s_hatL):
    blk = 1000
    spec = pl.BlockSpec((blk, K), lambda i: (i, 0))
    return pl.pallas_call(
        _softmax2_body,
        grid=(N // blk,),
        in_specs=[spec, spec],
        out_specs=[spec, spec],
        out_shape=[
            jax.ShapeDtypeStruct((N, K), jnp.float32),
            jax.ShapeDtypeStruct((N, K), jnp.float32),
        ],
    )(s_hat0, s_hatL)


def _psi(x, edge_index, edge_attr, Wr, Wn, We, b):
    src = edge_index[0]
    dst = edge_index[1]
    msg = x[src] @ Wn + edge_attr @ We
    agg = jax.ops.segment_sum(msg, dst, num_segments=x.shape[0])
    return x @ Wr + agg + b


def kernel(x_s, edge_index_s, edge_attr_s, batch_s, x_t, edge_index_t,
           edge_attr_t, batch_t, W1r, W1n, W1e, b1, W2r, W2n, W2e, b2,
           M1, mb1, M2, mb2, r_s_steps):
    h_s = _psi(x_s, edge_index_s, edge_attr_s, W1r, W1n, W1e, b1)
    h_t = _psi(x_t, edge_index_t, edge_attr_t, W1r, W1n, W1e, b1)
    S_idx = _topk_idx(h_s.astype(jnp.bfloat16), h_t.astype(jnp.bfloat16))
    tmp_t = h_t[S_idx]
    S_hat = (h_s[:, None, :] * tmp_t).sum(axis=-1)
    S_hat0 = S_hat
    for step in range(NUM_STEPS):
        S = jax.nn.softmax(S_hat, axis=-1)
        r_s = r_s_steps[step]
        tmp = (r_s[:, None, :] * S[:, :, None]).reshape(N * K, -1)
        r_t = jax.ops.segment_sum(tmp, S_idx.reshape(N * K), num_segments=N)
        o_s = _psi(r_s, edge_index_s, edge_attr_s, W2r, W2n, W2e, b2)
        o_t = _psi(r_t, edge_index_t, edge_attr_t, W2r, W2n, W2e, b2)
        D = o_s[:, None, :] - o_t[S_idx]
        m = jax.nn.relu(D @ M1 + mb1) @ M2 + mb2
        S_hat = S_hat + m[..., 0]
    S_0, S_L = _softmax_pair(S_hat0, S_hat)
    return (S_0, S_L)


# speedup vs baseline: 1.7960x; 1.7960x over previous
import functools

import jax
import jax.numpy as jnp
from jax.experimental import pallas as pl
from jax.experimental.pallas import tpu as pltpu

N = 10000
F = 128
K = 10
NUM_STEPS = 2

MBLK = 400
NPAD = 10112
NCHUNK = NPAD // 128
NEG = -3.0e38


def _topk_body(hs_ref, ht_ref, idx_ref, s_ref):
    s = jax.lax.dot_general(hs_ref[...], ht_ref[...], (((1,), (1,)), ((), ())),
                            preferred_element_type=jnp.float32)
    col = jax.lax.broadcasted_iota(jnp.int32, (MBLK, NPAD), 1)
    s_ref[...] = jnp.where(col < N, s, NEG)

    def max_tile(t, m):
        tile = s_ref[:, pl.ds(t * 128, 128)]
        return jnp.maximum(m, jnp.max(tile, axis=1, keepdims=True))

    def loc_tile(t, best):
        tile = s_ref[:, pl.ds(t * 128, 128)]
        ti = jax.lax.broadcasted_iota(jnp.int32, (MBLK, 128), 1) + t * 128
        m, _ = best
        cand = jnp.min(jnp.where(tile == m, ti, jnp.int32(2**30)), axis=1,
                       keepdims=True)
        return (m, jnp.minimum(best[1], cand))

    def mask_tile(t, idx):
        tile = s_ref[:, pl.ds(t * 128, 128)]
        ti = jax.lax.broadcasted_iota(jnp.int32, (MBLK, 128), 1) + t * 128
        s_ref[:, pl.ds(t * 128, 128)] = jnp.where(ti == idx, NEG, tile)
        return idx

    cols = []
    for _ in range(K):
        m = jax.lax.fori_loop(0, NCHUNK, max_tile,
                              jnp.full((MBLK, 1), NEG, jnp.float32),
                              unroll=8)
        _, idx = jax.lax.fori_loop(0, NCHUNK, loc_tile,
                                   (m, jnp.full((MBLK, 1), 2**30, jnp.int32)),
                                   unroll=8)
        idx = jax.lax.fori_loop(0, NCHUNK, mask_tile, idx, unroll=8)
        cols.append(idx)
    idx_ref[...] = jnp.concatenate(cols, axis=1)


def _topk_idx(h_s_bf, h_t_bf):
    ht_pad = jnp.zeros((NPAD, F), jnp.bfloat16).at[:N].set(h_t_bf)
    return pl.pallas_call(
        _topk_body,
        grid=(N // MBLK,),
        in_specs=[
            pl.BlockSpec((MBLK, F), lambda i: (i, 0)),
            pl.BlockSpec((NPAD, F), lambda i: (0, 0)),
        ],
        out_specs=pl.BlockSpec((MBLK, K), lambda i: (i, 0)),
        out_shape=jax.ShapeDtypeStruct((N, K), jnp.int32),
        scratch_shapes=[pltpu.VMEM((MBLK, NPAD), jnp.float32)],
    )(h_s_bf, ht_pad)


def _softmax2_body(a_ref, b_ref, oa_ref, ob_ref):
    a = a_ref[...]
    b = b_ref[...]
    a = a - jnp.max(a, axis=-1, keepdims=True)
    ea = jnp.exp(a)
    oa_ref[...] = ea / jnp.sum(ea, axis=-1, keepdims=True)
    b = b - jnp.max(b, axis=-1, keepdims=True)
    eb = jnp.exp(b)
    ob_ref[...] = eb / jnp.sum(eb, axis=-1, keepdims=True)


def _softmax_pair(s_hat0, s_hatL):
    blk = 1000
    spec = pl.BlockSpec((blk, K), lambda i: (i, 0))
    return pl.pallas_call(
        _softmax2_body,
        grid=(N // blk,),
        in_specs=[spec, spec],
        out_specs=[spec, spec],
        out_shape=[
            jax.ShapeDtypeStruct((N, K), jnp.float32),
            jax.ShapeDtypeStruct((N, K), jnp.float32),
        ],
    )(s_hat0, s_hatL)


def _psi(x, edge_index, edge_attr, Wr, Wn, We, b):
    src = edge_index[0]
    dst = edge_index[1]
    msg = x[src] @ Wn + edge_attr @ We
    agg = jax.ops.segment_sum(msg, dst, num_segments=x.shape[0])
    return x @ Wr + agg + b


def kernel(x_s, edge_index_s, edge_attr_s, batch_s, x_t, edge_index_t,
           edge_attr_t, batch_t, W1r, W1n, W1e, b1, W2r, W2n, W2e, b2,
           M1, mb1, M2, mb2, r_s_steps):
    h_s = _psi(x_s, edge_index_s, edge_attr_s, W1r, W1n, W1e, b1)
    h_t = _psi(x_t, edge_index_t, edge_attr_t, W1r, W1n, W1e, b1)
    S_idx = _topk_idx(h_s.astype(jnp.bfloat16), h_t.astype(jnp.bfloat16))
    tmp_t = h_t[S_idx]
    S_hat = (h_s[:, None, :] * tmp_t).sum(axis=-1)
    S_hat0 = S_hat
    for step in range(NUM_STEPS):
        S = jax.nn.softmax(S_hat, axis=-1)
        r_s = r_s_steps[step]
        tmp = (r_s[:, None, :] * S[:, :, None]).reshape(N * K, -1)
        r_t = jax.ops.segment_sum(tmp, S_idx.reshape(N * K), num_segments=N)
        o_s = _psi(r_s, edge_index_s, edge_attr_s, W2r, W2n, W2e, b2)
        o_t = _psi(r_t, edge_index_t, edge_attr_t, W2r, W2n, W2e, b2)
        D = o_s[:, None, :] - o_t[S_idx]
        m = jax.nn.relu(D @ M1 + mb1) @ M2 + mb2
        S_hat = S_hat + m[..., 0]
    S_0, S_L = _softmax_pair(S_hat0, S_hat)
    return (S_0, S_L)


# speedup vs baseline: 1.7961x; 1.0001x over previous
import functools

import jax
import jax.numpy as jnp
from jax import lax
from jax.experimental import pallas as pl
from jax.experimental.pallas import tpu as pltpu

N = 10000
E = 160000
F = 128
R = 32
K = 10
NUM_STEPS = 2

MBLK = 400
NPAD = 10112
NCHUNK = NPAD // 128
NEG = -3.0e38


def _topk_body(hs_ref, ht_ref, idx_ref, s_ref):
    s = jax.lax.dot_general(hs_ref[...], ht_ref[...], (((1,), (1,)), ((), ())),
                            preferred_element_type=jnp.float32)
    col = jax.lax.broadcasted_iota(jnp.int32, (MBLK, NPAD), 1)
    s_ref[...] = jnp.where(col < N, s, NEG)

    def max_tile(t, m):
        tile = s_ref[:, pl.ds(t * 128, 128)]
        return jnp.maximum(m, jnp.max(tile, axis=1, keepdims=True))

    def loc_tile(t, best):
        tile = s_ref[:, pl.ds(t * 128, 128)]
        ti = jax.lax.broadcasted_iota(jnp.int32, (MBLK, 128), 1) + t * 128
        m, _ = best
        cand = jnp.min(jnp.where(tile == m, ti, jnp.int32(2**30)), axis=1,
                       keepdims=True)
        return (m, jnp.minimum(best[1], cand))

    def mask_tile(t, idx):
        tile = s_ref[:, pl.ds(t * 128, 128)]
        ti = jax.lax.broadcasted_iota(jnp.int32, (MBLK, 128), 1) + t * 128
        s_ref[:, pl.ds(t * 128, 128)] = jnp.where(ti == idx, NEG, tile)
        return idx

    cols = []
    for _ in range(K):
        m = jax.lax.fori_loop(0, NCHUNK, max_tile,
                              jnp.full((MBLK, 1), NEG, jnp.float32),
                              unroll=8)
        _, idx = jax.lax.fori_loop(0, NCHUNK, loc_tile,
                                   (m, jnp.full((MBLK, 1), 2**30, jnp.int32)),
                                   unroll=8)
        idx = jax.lax.fori_loop(0, NCHUNK, mask_tile, idx, unroll=8)
        cols.append(idx)
    idx_ref[...] = jnp.concatenate(cols, axis=1)


def _topk_idx(h_s_bf, h_t_bf):
    ht_pad = jnp.zeros((NPAD, F), jnp.bfloat16).at[:N].set(h_t_bf)
    return pl.pallas_call(
        _topk_body,
        grid=(N // MBLK,),
        in_specs=[
            pl.BlockSpec((MBLK, F), lambda i: (i, 0)),
            pl.BlockSpec((NPAD, F), lambda i: (0, 0)),
        ],
        out_specs=pl.BlockSpec((MBLK, K), lambda i: (i, 0)),
        out_shape=jax.ShapeDtypeStruct((N, K), jnp.int32),
        scratch_shapes=[pltpu.VMEM((MBLK, NPAD), jnp.float32)],
    )(h_s_bf, ht_pad)


def _softmax2_body(a_ref, b_ref, oa_ref, ob_ref):
    a = a_ref[...]
    b = b_ref[...]
    a = a - jnp.max(a, axis=-1, keepdims=True)
    ea = jnp.exp(a)
    oa_ref[...] = ea / jnp.sum(ea, axis=-1, keepdims=True)
    b = b - jnp.max(b, axis=-1, keepdims=True)
    eb = jnp.exp(b)
    ob_ref[...] = eb / jnp.sum(eb, axis=-1, keepdims=True)


def _softmax_pair(s_hat0, s_hatL):
    blk = 1000
    spec = pl.BlockSpec((blk, K), lambda i: (i, 0))
    return pl.pallas_call(
        _softmax2_body,
        grid=(N // blk,),
        in_specs=[spec, spec],
        out_specs=[spec, spec],
        out_shape=[
            jax.ShapeDtypeStruct((N, K), jnp.float32),
            jax.ShapeDtypeStruct((N, K), jnp.float32),
        ],
    )(s_hat0, s_hatL)


def _psi(x, src, dst, edge_attr, Wr, Wn, We, b):
    msg = x[src] @ Wn + edge_attr @ We
    agg = jax.ops.segment_sum(msg, dst, num_segments=x.shape[0])
    return x @ Wr + agg + b


def kernel(x_s, edge_index_s, edge_attr_s, batch_s, x_t, edge_index_t,
           edge_attr_t, batch_t, W1r, W1n, W1e, b1, W2r, W2n, W2e, b2,
           M1, mb1, M2, mb2, r_s_steps):
    h_s = _psi(x_s, edge_index_s[0], edge_index_s[1], edge_attr_s,
               W1r, W1n, W1e, b1)
    h_t = _psi(x_t, edge_index_t[0], edge_index_t[1], edge_attr_t,
               W1r, W1n, W1e, b1)

    S_idx = _topk_idx(h_s.astype(jnp.bfloat16), h_t.astype(jnp.bfloat16))

    tmp_t = h_t[S_idx]
    S_hat = (h_s[:, None, :] * tmp_t).sum(axis=-1)
    S_hat0 = S_hat

    for step in range(NUM_STEPS):
        S = jax.nn.softmax(S_hat, axis=-1)
        r_s = r_s_steps[step]
        tmp = (r_s[:, None, :] * S[:, :, None]).reshape(N * K, R)
        r_t = jax.ops.segment_sum(tmp, S_idx.reshape(N * K), num_segments=N)
        o_s = _psi(r_s, edge_index_s[0], edge_index_s[1], edge_attr_s,
                   W2r, W2n, W2e, b2)
        o_t = _psi(r_t, edge_index_t[0], edge_index_t[1], edge_attr_t,
                   W2r, W2n, W2e, b2)
        D = o_s[:, None, :] - o_t[S_idx]
        m = jax.nn.relu(D @ M1 + mb1) @ M2 + mb2
        S_hat = S_hat + m[..., 0]

    S_0, S_L = _softmax_pair(S_hat0, S_hat)
    return (S_0, S_L)


# speedup vs baseline: 2.0170x; 1.1230x over previous
import functools

import jax
import jax.numpy as jnp
from jax import lax
from jax.experimental import pallas as pl
from jax.experimental.pallas import tpu as pltpu
from jax.experimental.pallas import tpu_sc as plsc

N = 10000
E = 160000
F = 128
R = 32
K = 10
NUM_STEPS = 2

NUNITS = E // 128
NACC = 10240
NPG = NACC // 16
NKPAD = 100096
GUNITS = NKPAD // 128

MBLK = 400
NPAD = 10112
NCHUNK = NPAD // 128
NEG = -3.0e38

_MESH = plsc.VectorSubcoreMesh(core_axis_name="c", subcore_axis_name="s")


def _psi2_seg(P_flat, srcm, dstm, z32):
    @functools.partial(
        pl.kernel,
        out_type=jax.ShapeDtypeStruct((2 * NACC, R), jnp.float32),
        mesh=_MESH,
        compiler_params=pltpu.CompilerParams(use_tc_tiling_on_sc=False),
        scratch_types=[pltpu.VMEM_SHARED((NACC, R), jnp.float32),
                       pltpu.VMEM((128,), jnp.int32),
                       pltpu.VMEM((128,), jnp.int32),
                       pltpu.VMEM((128, R), jnp.float32),
                       pltpu.SemaphoreType.DMA],
    )
    def k(P_hbm, src_hbm, dst_hbm, z32_hbm, out_hbm,
          acc, src_v, dst_v, rows_v, sem):
        c = lax.axis_index("c")
        s = lax.axis_index("s")
        pltpu.sync_copy(z32_hbm, acc.at[pl.ds(s * NPG, NPG)])
        plsc.subcore_barrier()

        def body(i, carry):
            u = s + 16 * i

            @pl.when(u < NUNITS)
            def _():
                row = c * NUNITS + u
                pltpu.sync_copy(src_hbm.at[pl.ds(row * 128, 128)], src_v)
                pltpu.sync_copy(dst_hbm.at[pl.ds(row * 128, 128)], dst_v)
                pltpu.async_copy(P_hbm.at[src_v], rows_v, sem).wait()
                pltpu.sync_copy(rows_v, acc.at[dst_v], add=True)
            return carry

        lax.fori_loop(0, (NUNITS + 15) // 16, body, 0)
        plsc.subcore_barrier()
        pltpu.sync_copy(acc.at[pl.ds(s * NPG, NPG)],
                        out_hbm.at[pl.ds(c * NACC + s * NPG, NPG)])

    return k(P_flat, srcm, dstm, z32)


def _scatter_rt(tmp_pad, idxm, z32):
    @functools.partial(
        pl.kernel,
        out_type=jax.ShapeDtypeStruct((2 * NACC, R), jnp.float32),
        mesh=_MESH,
        compiler_params=pltpu.CompilerParams(use_tc_tiling_on_sc=False),
        scratch_types=[pltpu.VMEM_SHARED((NACC, R), jnp.float32),
                       pltpu.VMEM((128,), jnp.int32),
                       pltpu.VMEM((128, R), jnp.float32),
                       pltpu.SemaphoreType.DMA],
    )
    def k(tmp_hbm, idx_hbm, z32_hbm, out_hbm, acc, idx_v, rows_v, sem):
        c = lax.axis_index("c")
        s = lax.axis_index("s")
        w = s * 2 + c
        pltpu.sync_copy(z32_hbm, acc.at[pl.ds(s * NPG, NPG)])
        plsc.subcore_barrier()

        def body(i, carry):
            u = w + 32 * i

            @pl.when(u < GUNITS)
            def _():
                pltpu.sync_copy(idx_hbm.at[pl.ds(u * 128, 128)], idx_v)
                pltpu.sync_copy(tmp_hbm.at[pl.ds(u * 128, 128)], rows_v)
                pltpu.sync_copy(rows_v, acc.at[idx_v], add=True)
            return carry

        lax.fori_loop(0, (GUNITS + 31) // 32, body, 0)
        plsc.subcore_barrier()
        pltpu.sync_copy(acc.at[pl.ds(s * NPG, NPG)],
                        out_hbm.at[pl.ds(c * NACC + s * NPG, NPG)])

    return k(tmp_pad, idxm, z32)


def _topk_body(hs_ref, ht_ref, idx_ref, s_ref):
    s = jax.lax.dot_general(hs_ref[...], ht_ref[...], (((1,), (1,)), ((), ())),
                            preferred_element_type=jnp.float32)
    col = jax.lax.broadcasted_iota(jnp.int32, (MBLK, NPAD), 1)
    s_ref[...] = jnp.where(col < N, s, NEG)

    def max_tile(t, m):
        tile = s_ref[:, pl.ds(t * 128, 128)]
        return jnp.maximum(m, jnp.max(tile, axis=1, keepdims=True))

    def loc_tile(t, best):
        tile = s_ref[:, pl.ds(t * 128, 128)]
        ti = jax.lax.broadcasted_iota(jnp.int32, (MBLK, 128), 1) + t * 128
        m, _ = best
        cand = jnp.min(jnp.where(tile == m, ti, jnp.int32(2**30)), axis=1,
                       keepdims=True)
        return (m, jnp.minimum(best[1], cand))

    def mask_tile(t, idx):
        tile = s_ref[:, pl.ds(t * 128, 128)]
        ti = jax.lax.broadcasted_iota(jnp.int32, (MBLK, 128), 1) + t * 128
        s_ref[:, pl.ds(t * 128, 128)] = jnp.where(ti == idx, NEG, tile)
        return idx

    cols = []
    for _ in range(K):
        m = jax.lax.fori_loop(0, NCHUNK, max_tile,
                              jnp.full((MBLK, 1), NEG, jnp.float32),
                              unroll=8)
        _, idx = jax.lax.fori_loop(0, NCHUNK, loc_tile,
                                   (m, jnp.full((MBLK, 1), 2**30, jnp.int32)),
                                   unroll=8)
        idx = jax.lax.fori_loop(0, NCHUNK, mask_tile, idx, unroll=8)
        cols.append(idx)
    idx_ref[...] = jnp.concatenate(cols, axis=1)


def _topk_idx(h_s_bf, h_t_bf):
    ht_pad = jnp.zeros((NPAD, F), jnp.bfloat16).at[:N].set(h_t_bf)
    return pl.pallas_call(
        _topk_body,
        grid=(N // MBLK,),
        in_specs=[
            pl.BlockSpec((MBLK, F), lambda i: (i, 0)),
            pl.BlockSpec((NPAD, F), lambda i: (0, 0)),
        ],
        out_specs=pl.BlockSpec((MBLK, K), lambda i: (i, 0)),
        out_shape=jax.ShapeDtypeStruct((N, K), jnp.int32),
        scratch_shapes=[pltpu.VMEM((MBLK, NPAD), jnp.float32)],
    )(h_s_bf, ht_pad)


def _softmax2_body(a_ref, b_ref, oa_ref, ob_ref):
    a = a_ref[...]
    b = b_ref[...]
    a = a - jnp.max(a, axis=-1, keepdims=True)
    ea = jnp.exp(a)
    oa_ref[...] = ea / jnp.sum(ea, axis=-1, keepdims=True)
    b = b - jnp.max(b, axis=-1, keepdims=True)
    eb = jnp.exp(b)
    ob_ref[...] = eb / jnp.sum(eb, axis=-1, keepdims=True)


def _softmax_pair(s_hat0, s_hatL):
    blk = 1000
    spec = pl.BlockSpec((blk, K), lambda i: (i, 0))
    return pl.pallas_call(
        _softmax2_body,
        grid=(N // blk,),
        in_specs=[spec, spec],
        out_specs=[spec, spec],
        out_shape=[
            jax.ShapeDtypeStruct((N, K), jnp.float32),
            jax.ShapeDtypeStruct((N, K), jnp.float32),
        ],
    )(s_hat0, s_hatL)


def _attr_term(C4, We):
    We_f = We.astype(jnp.bfloat16).astype(jnp.float32)
    out = C4[:, 0:1] * We_f[0:1, :]
    for d in range(1, 4):
        out = out + C4[:, d:d + 1] * We_f[d:d + 1, :]
    return out


def _psi(x, src, dst, edge_attr, Wr, Wn, We, b):
    msg = x[src] @ Wn + edge_attr @ We
    agg = jax.ops.segment_sum(msg, dst, num_segments=x.shape[0])
    return x @ Wr + agg + b


def kernel(x_s, edge_index_s, edge_attr_s, batch_s, x_t, edge_index_t,
           edge_attr_t, batch_t, W1r, W1n, W1e, b1, W2r, W2n, W2e, b2,
           M1, mb1, M2, mb2, r_s_steps):
    h_s = _psi(x_s, edge_index_s[0], edge_index_s[1], edge_attr_s,
               W1r, W1n, W1e, b1)
    h_t = _psi(x_t, edge_index_t[0], edge_index_t[1], edge_attr_t,
               W1r, W1n, W1e, b1)

    S_idx = _topk_idx(h_s.astype(jnp.bfloat16), h_t.astype(jnp.bfloat16))

    tmp_t = h_t[S_idx]
    S_hat = (h_s[:, None, :] * tmp_t).sum(axis=-1)
    S_hat0 = S_hat

    srcm = jnp.concatenate([edge_index_s[0], edge_index_t[0] + N])
    dstm = jnp.concatenate([edge_index_s[1], edge_index_t[1]])
    z32 = jnp.zeros((NPG, R), jnp.float32)
    idxm = jnp.zeros((NKPAD,), jnp.int32).at[:N * K].set(S_idx.reshape(-1))
    ea_bf_s = edge_attr_s.astype(jnp.bfloat16).astype(jnp.float32)
    ea_bf_t = edge_attr_t.astype(jnp.bfloat16).astype(jnp.float32)
    C4_s = jax.ops.segment_sum(ea_bf_s, edge_index_s[1], num_segments=N)
    C4_t = jax.ops.segment_sum(ea_bf_t, edge_index_t[1], num_segments=N)

    for step in range(NUM_STEPS):
        S = jax.nn.softmax(S_hat, axis=-1)
        r_s = r_s_steps[step]
        tmp = (r_s[:, None, :] * S[:, :, None]).reshape(N * K, R)
        tmp_pad = jnp.zeros((NKPAD, R), jnp.float32).at[:N * K].set(tmp)
        rt_par = _scatter_rt(tmp_pad, idxm, z32)
        r_t = rt_par[:N] + rt_par[NACC:NACC + N]
        P2 = jnp.concatenate([r_s @ W2n, r_t @ W2n])
        agg2 = _psi2_seg(P2, srcm, dstm, z32)
        o_s = r_s @ W2r + (agg2[:N] + _attr_term(C4_s, W2e)) + b2
        o_t = r_t @ W2r + (agg2[NACC:NACC + N] + _attr_term(C4_t, W2e)) + b2
        D = o_s[:, None, :] - o_t[S_idx]
        m = jax.nn.relu(D @ M1 + mb1) @ M2 + mb2
        S_hat = S_hat + m[..., 0]

    S_0, S_L = _softmax_pair(S_hat0, S_hat)
    return (S_0, S_L)
